# baseline (speedup 1.0000x reference)
import functools

import jax
import jax.numpy as jnp
from jax import lax
from jax.experimental import pallas as pl
from jax.experimental.pallas import tpu as pltpu
from jax.experimental.pallas import tpu_sc as plsc


def _worker_mesh():
    return plsc.VectorSubcoreMesh(core_axis_name="c", subcore_axis_name="s")


def _build_phase1(V, D, NC, NS):
    NW = NC * NS
    assert D == 64 and V % 2 == 0
    NFULL = V // 128
    TAIL = V - NFULL * 128
    assert TAIL in (0, 64)
    n_i = (NFULL + NW - 1) // NW

    @functools.partial(
        pl.kernel,
        mesh=_worker_mesh(),
        compiler_params=pltpu.CompilerParams(needs_layout_passes=False),
        out_type=jax.ShapeDtypeStruct((V // 2, 2 * D), jnp.float32),
        scratch_types=[
            pltpu.VMEM((D, 128), jnp.float32),
            pltpu.VMEM((D, 128), jnp.float32),
        ],
    )
    def tr(table_t, tail_pairs, pairs, tb, tt):
        wid = lax.axis_index("s") * NC + lax.axis_index("c")
        iota = lax.iota(jnp.int32, 16)

        def transpose_block(np_rows):
            @pl.loop(0, np_rows)
            def _p(p):
                two_p = 2 * p
                for qg in range(8):
                    q0 = qg * 16
                    h = 0 if q0 < D else 1
                    d0 = q0 - D * h
                    dvec = iota + d0
                    mvec = jnp.broadcast_to(two_p + h, (16,)).astype(jnp.int32)
                    tt[p, pl.ds(q0, 16)] = plsc.load_gather(tb, [dvec, mvec])

        @pl.loop(0, n_i)
        def _i(i):
            blk = i * NW + wid

            @pl.when(blk < NFULL)
            def _full():
                t0 = blk * 128
                pltpu.sync_copy(table_t.at[:, pl.ds(t0, 128)], tb)
                transpose_block(64)
                pltpu.sync_copy(tt, pairs.at[pl.ds(blk * 64, 64), :])

        if TAIL:
            @pl.when(wid == NW - 1)
            def _tail():
                pltpu.sync_copy(
                    tail_pairs, pairs.at[pl.ds(NFULL * 64, TAIL // 2), :]
                )

    return tr


def _build_phase2(B, L, V, D, NC, NS):
    NW = NC * NS
    assert B % NW == 0
    bw = B // NW
    CH = 256
    assert bw % CH == 0
    ncb = bw // CH

    @functools.partial(
        pl.kernel,
        mesh=_worker_mesh(),
        compiler_params=pltpu.CompilerParams(needs_layout_passes=False),
        out_type=jax.ShapeDtypeStruct((L, D, B), jnp.float32),
        scratch_types=[
            pltpu.VMEM((CH,), jnp.int32),
            pltpu.VMEM((CH,), jnp.int32),
            pltpu.VMEM((CH,), jnp.int32),
            pltpu.VMEM((CH, 2 * D), jnp.float32),
            pltpu.VMEM((D, CH), jnp.float32),
            pltpu.SemaphoreType.DMA,
        ],
    )
    def ga(idx_t, pairs, out_t, idxb, pairv, colsel, rows_v, ot, semg):
        wid = lax.axis_index("s") * NC + lax.axis_index("c")
        b_base = wid * bw
        iota = lax.iota(jnp.int32, 16)

        @pl.loop(0, L * ncb)
        def _c(t):
            l = t // ncb
            b0 = b_base + (t % ncb) * CH
            pltpu.sync_copy(idx_t.at[l, pl.ds(b0, CH)], idxb)

            @pl.loop(0, CH // 16)
            def _g(g):
                v = idxb[pl.ds(g * 16, 16)]
                pairv[pl.ds(g * 16, 16)] = lax.shift_right_logical(v, 1)
                colsel[pl.ds(g * 16, 16)] = lax.shift_left(v & 1, 6)

            pltpu.async_copy(pairs.at[pairv], rows_v, semg).wait()

            @pl.loop(0, CH // 16)
            def _g2(g):
                rowi = iota + g * 16
                cbase = colsel[pl.ds(g * 16, 16)]
                for d in range(D):
                    ot[d, pl.ds(g * 16, 16)] = plsc.load_gather(
                        rows_v, [rowi, cbase + d]
                    )

            pltpu.sync_copy(ot, out_t.at[l, :, pl.ds(b0, CH)])

    return ga


def kernel(input, table):
    B, L = input.shape
    V, D = table.shape
    try:
        info = plsc.get_sparse_core_info()
        NC, NS = info.num_cores, info.num_subcores
    except Exception:
        NC, NS = 2, 16
    idx_t = input.T.astype(jnp.int32)
    table_t = table.T
    nfull = V // 128
    tail_pairs = table[nfull * 128:, :].reshape(-1, 2 * D)
    pairs = _build_phase1(V, D, NC, NS)(table_t, tail_pairs)
    out_t = _build_phase2(B, L, V, D, NC, NS)(idx_t, pairs)
    return out_t.transpose(2, 0, 1)


# speedup vs baseline: 1.6379x; 1.6379x over previous
import functools

import jax
import jax.numpy as jnp
from jax import lax
from jax.experimental import pallas as pl
from jax.experimental.pallas import tpu as pltpu
from jax.experimental.pallas import tpu_sc as plsc


def _build_gather(B, L, V, D, NC, NS):
    NW = NC * NS
    assert B % NW == 0 and D == 64
    bw = B // NW
    CH = 256
    assert bw % CH == 0
    ncb = bw // CH
    T = L * ncb
    assert T % 2 == 0

    @functools.partial(
        pl.kernel,
        mesh=plsc.VectorSubcoreMesh(core_axis_name="c", subcore_axis_name="s"),
        compiler_params=pltpu.CompilerParams(needs_layout_passes=False),
        out_type=jax.ShapeDtypeStruct((L, D, B), jnp.float32),
        scratch_types=[
            pltpu.VMEM((2, CH), jnp.int32),
            pltpu.VMEM((CH,), jnp.int32),
            pltpu.VMEM((CH,), jnp.int32),
            pltpu.VMEM((2, CH), jnp.int32),
            pltpu.VMEM((2, CH, 2 * D), jnp.float32),
            pltpu.VMEM((2, D, CH), jnp.float32),
        ]
        + [pltpu.SemaphoreType.DMA] * 6,
    )
    def ga(idx_t, pairs, out_t, idxb, pairv0, pairv1, colsel, rows_v, ot, *sems):
        pairv = (pairv0, pairv1)
        semi = sems[0:2]
        semg = sems[2:4]
        semo = sems[4:6]
        wid = lax.axis_index("s") * NC + lax.axis_index("c")
        b_base = wid * bw
        iota = lax.iota(jnp.int32, 16)

        def idx_src(t):
            return idx_t.at[t // ncb, pl.ds(b_base + (t % ncb) * CH, CH)]

        def out_dst(t):
            return out_t.at[t // ncb, :, pl.ds(b_base + (t % ncb) * CH, CH)]

        def compute_and_fire(t, b):
            @pl.loop(0, CH // 16, unroll=4)
            def _g(g):
                v = idxb[b, pl.ds(g * 16, 16)]
                pairv[b][pl.ds(g * 16, 16)] = lax.shift_right_logical(v, 1)
                colsel[b, pl.ds(g * 16, 16)] = lax.shift_left(v & 1, 6)

            pltpu.async_copy(pairs.at[pairv[b]], rows_v.at[b], semg[b])

        def select_transpose(b):
            @pl.loop(0, CH // 16)
            def _g2(g):
                rowi = iota + g * 16
                cbase = colsel[b, pl.ds(g * 16, 16)]
                for d in range(D):
                    ot[b, d, pl.ds(g * 16, 16)] = plsc.load_gather(
                        rows_v.at[b], [rowi, cbase + d]
                    )

        pltpu.sync_copy(idx_src(0), idxb.at[0])
        compute_and_fire(0, 0)

        @pl.loop(0, T, step=2)
        def _outer(t0):
            for j in range(2):
                t = t0 + j
                b = j
                b1 = 1 - j

                def stage_next():
                    pltpu.async_copy(idx_src(t + 1), idxb.at[b1], semi[b1])

                if j == 0:
                    stage_next()
                else:
                    @pl.when(t + 1 < T)
                    def _():
                        stage_next()

                pltpu.make_async_copy(
                    pairs.at[pairv[b]], rows_v.at[b], semg[b]
                ).wait()

                @pl.when(t >= 2)
                def _():
                    pltpu.make_async_copy(
                        ot.at[b], out_dst(t - 2), semo[b]
                    ).wait()

                select_transpose(b)
                pltpu.async_copy(ot.at[b], out_dst(t), semo[b])

                def drain_and_fire():
                    pltpu.make_async_copy(
                        idx_src(t + 1), idxb.at[b1], semi[b1]
                    ).wait()
                    compute_and_fire(t + 1, b1)

                if j == 0:
                    drain_and_fire()
                else:
                    @pl.when(t + 1 < T)
                    def _():
                        drain_and_fire()

        for j in range(2):
            t = T - 2 + j
            pltpu.make_async_copy(ot.at[j], out_dst(t), semo[j]).wait()

    return ga


def kernel(input, table):
    B, L = input.shape
    V, D = table.shape
    try:
        info = plsc.get_sparse_core_info()
        NC, NS = info.num_cores, info.num_subcores
    except Exception:
        NC, NS = 2, 16
    idx_t = input.T.astype(jnp.int32)
    pairs = jnp.reshape(table, (V // 2, 2 * D))
    out_t = _build_gather(B, L, V, D, NC, NS)(idx_t, pairs)
    return out_t.transpose(2, 0, 1)


# speedup vs baseline: 1.8480x; 1.1283x over previous
import functools

import jax
import jax.numpy as jnp
from jax import lax
from jax.experimental import pallas as pl
from jax.experimental.pallas import tpu as pltpu
from jax.experimental.pallas import tpu_sc as plsc


def _build_gather(B, L, V, D, NC, NS):
    NW = NC * NS
    assert B % NW == 0 and D == 64
    bw = B // NW
    CH = 256
    assert bw % CH == 0
    ncb = bw // CH
    T = L * ncb
    assert T % 2 == 0

    @functools.partial(
        pl.kernel,
        mesh=plsc.VectorSubcoreMesh(core_axis_name="c", subcore_axis_name="s"),
        compiler_params=pltpu.CompilerParams(needs_layout_passes=False),
        out_type=jax.ShapeDtypeStruct((L, D, B), jnp.float32),
        scratch_types=[
            pltpu.VMEM((2, CH), jnp.int32),
            pltpu.VMEM((CH,), jnp.int32),
            pltpu.VMEM((CH,), jnp.int32),
            pltpu.VMEM((2, CH), jnp.int32),
            pltpu.VMEM((2, CH, 2 * D), jnp.float32),
            pltpu.VMEM((2, D, CH), jnp.float32),
            pltpu.VMEM((16, 16), jnp.int32),
        ]
        + [pltpu.SemaphoreType.DMA] * 6,
    )
    def ga(idx_t, pairs, out_t, idxb, pairv0, pairv1, colsel, rows_v, ot, rot_v, *sems):
        pairv = (pairv0, pairv1)
        semi = sems[0:2]
        semg = sems[2:4]
        semo = sems[4:6]
        wid = lax.axis_index("s") * NC + lax.axis_index("c")
        b_base = wid * bw
        iota = lax.iota(jnp.int32, 16)

        def idx_src(t):
            return idx_t.at[t // ncb, pl.ds(b_base + (t % ncb) * CH, CH)]

        def out_dst(t):
            return out_t.at[t // ncb, :, pl.ds(b_base + (t % ncb) * CH, CH)]

        def compute_and_fire(t, b):
            @pl.loop(0, CH // 16, unroll=4)
            def _g(g):
                v = idxb[b, pl.ds(g * 16, 16)]
                pairv[b][pl.ds(g * 16, 16)] = lax.shift_right_logical(v, 1)
                colsel[b, pl.ds(g * 16, 16)] = lax.shift_left(v & 1, 6)

            pltpu.async_copy(pairs.at[pairv[b]], rows_v.at[b], semg[b])

        def select_transpose(b):
            @pl.loop(0, 16)
            def _k(k):
                rot = rot_v[k, :]
                for di in range(D // 16):
                    drot = rot + di * 16
                    for g in range(CH // 16):
                        tokv = iota + g * 16
                        cbase = colsel[b, pl.ds(g * 16, 16)]
                        val = plsc.load_gather(
                            rows_v.at[b], [tokv, cbase + drot]
                        )
                        plsc.store_scatter(ot.at[b], [drot, tokv], val)

        for k in range(16):
            rot_v[k, :] = (iota + k) & 15

        pltpu.sync_copy(idx_src(0), idxb.at[0])
        compute_and_fire(0, 0)

        @pl.loop(0, T, step=2)
        def _outer(t0):
            for j in range(2):
                t = t0 + j
                b = j
                b1 = 1 - j

                def stage_next():
                    pltpu.async_copy(idx_src(t + 1), idxb.at[b1], semi[b1])

                if j == 0:
                    stage_next()
                else:
                    @pl.when(t + 1 < T)
                    def _():
                        stage_next()

                pltpu.make_async_copy(
                    pairs.at[pairv[b]], rows_v.at[b], semg[b]
                ).wait()

                @pl.when(t >= 2)
                def _():
                    pltpu.make_async_copy(
                        ot.at[b], out_dst(t - 2), semo[b]
                    ).wait()

                select_transpose(b)
                pltpu.async_copy(ot.at[b], out_dst(t), semo[b])

                def drain_and_fire():
                    pltpu.make_async_copy(
                        idx_src(t + 1), idxb.at[b1], semi[b1]
                    ).wait()
                    compute_and_fire(t + 1, b1)

                if j == 0:
                    drain_and_fire()
                else:
                    @pl.when(t + 1 < T)
                    def _():
                        drain_and_fire()

        for j in range(2):
            t = T - 2 + j
            pltpu.make_async_copy(ot.at[j], out_dst(t), semo[j]).wait()

    return ga


def kernel(input, table):
    B, L = input.shape
    V, D = table.shape
    try:
        info = plsc.get_sparse_core_info()
        NC, NS = info.num_cores, info.num_subcores
    except Exception:
        NC, NS = 2, 16
    idx_t = input.T.astype(jnp.int32)
    pairs = jnp.reshape(table, (V // 2, 2 * D))
    out_t = _build_gather(B, L, V, D, NC, NS)(idx_t, pairs)
    return out_t.transpose(2, 0, 1)


# speedup vs baseline: 2.1814x; 1.1804x over previous
import functools

import jax
import jax.numpy as jnp
from jax import lax
from jax.experimental import pallas as pl
from jax.experimental.pallas import tpu as pltpu
from jax.experimental.pallas import tpu_sc as plsc


def _build_gather(B, L, V, D, NC, NS):
    NW = NC * NS
    assert B % NW == 0 and D == 64
    bw = B // NW
    CH = 256
    assert bw % CH == 0
    ncb = bw // CH
    T = L * ncb
    assert T % 2 == 0

    @functools.partial(
        pl.kernel,
        mesh=plsc.VectorSubcoreMesh(core_axis_name="c", subcore_axis_name="s"),
        compiler_params=pltpu.CompilerParams(needs_layout_passes=False),
        out_type=jax.ShapeDtypeStruct((L, D, B), jnp.float32),
        scratch_types=[
            pltpu.VMEM((2, CH), jnp.int32),
            pltpu.VMEM((CH,), jnp.int32),
            pltpu.VMEM((CH,), jnp.int32),
            pltpu.VMEM((2, CH), jnp.int32),
            pltpu.VMEM((2, CH, 2 * D), jnp.float32),
            pltpu.VMEM((2, D, CH), jnp.float32),
            pltpu.VMEM((16, 16), jnp.int32),
        ]
        + [pltpu.SemaphoreType.DMA] * 6,
    )
    def ga(idx_t, pairs, out_t, idxb, pairv0, pairv1, colsel, rows_v, ot, rot_v, *sems):
        pairv = (pairv0, pairv1)
        semi = sems[0:2]
        semg = sems[2:4]
        semo = sems[4:6]
        wid = lax.axis_index("s") * NC + lax.axis_index("c")
        b_base = wid * bw
        iota = lax.iota(jnp.int32, 16)

        def idx_src(t):
            return idx_t.at[t // ncb, pl.ds(b_base + (t % ncb) * CH, CH)]

        def out_dst(t):
            return out_t.at[t // ncb, :, pl.ds(b_base + (t % ncb) * CH, CH)]

        def compute_and_fire(t, b):
            @plsc.parallel_loop(0, CH // 16, unroll=4)
            def _g(g):
                v = idxb[b, pl.ds(g * 16, 16)]
                pairv[b][pl.ds(g * 16, 16)] = lax.shift_right_logical(v, 1)
                colsel[b, pl.ds(g * 16, 16)] = lax.shift_left(v & 1, 6)

            pltpu.async_copy(pairs.at[pairv[b]], rows_v.at[b], semg[b])

        def select_transpose(b):
            @plsc.parallel_loop(0, 16, unroll=2)
            def _k(k):
                rot = rot_v[k, :]
                for di in range(D // 16):
                    drot = rot + di * 16
                    for g in range(CH // 16):
                        tokv = iota + g * 16
                        cbase = colsel[b, pl.ds(g * 16, 16)]
                        val = plsc.load_gather(
                            rows_v.at[b], [tokv, cbase + drot]
                        )
                        plsc.store_scatter(ot.at[b], [drot, tokv], val)

        for k in range(16):
            rot_v[k, :] = (iota + k) & 15

        pltpu.sync_copy(idx_src(0), idxb.at[0])
        compute_and_fire(0, 0)

        @pl.loop(0, T, step=2)
        def _outer(t0):
            for j in range(2):
                t = t0 + j
                b = j
                b1 = 1 - j

                def stage_next():
                    pltpu.async_copy(idx_src(t + 1), idxb.at[b1], semi[b1])

                if j == 0:
                    stage_next()
                else:
                    @pl.when(t + 1 < T)
                    def _():
                        stage_next()

                pltpu.make_async_copy(
                    pairs.at[pairv[b]], rows_v.at[b], semg[b]
                ).wait()

                @pl.when(t >= 2)
                def _():
                    pltpu.make_async_copy(
                        ot.at[b], out_dst(t - 2), semo[b]
                    ).wait()

                select_transpose(b)
                pltpu.async_copy(ot.at[b], out_dst(t), semo[b])

                def drain_and_fire():
                    pltpu.make_async_copy(
                        idx_src(t + 1), idxb.at[b1], semi[b1]
                    ).wait()
                    compute_and_fire(t + 1, b1)

                if j == 0:
                    drain_and_fire()
                else:
                    @pl.when(t + 1 < T)
                    def _():
                        drain_and_fire()

        for j in range(2):
            t = T - 2 + j
            pltpu.make_async_copy(ot.at[j], out_dst(t), semo[j]).wait()

    return ga


def kernel(input, table):
    B, L = input.shape
    V, D = table.shape
    try:
        info = plsc.get_sparse_core_info()
        NC, NS = info.num_cores, info.num_subcores
    except Exception:
        NC, NS = 2, 16
    idx_t = input.T.astype(jnp.int32)
    pairs = jnp.reshape(table, (V // 2, 2 * D))
    out_t = _build_gather(B, L, V, D, NC, NS)(idx_t, pairs)
    return out_t.transpose(2, 0, 1)


# speedup vs baseline: 3.3473x; 1.5345x over previous
import functools

import jax
import jax.numpy as jnp
from jax import lax
from jax.experimental import pallas as pl
from jax.experimental.pallas import tpu as pltpu
from jax.experimental.pallas import tpu_sc as plsc


def _build_gather(B, L, V, D, NC, NS):
    NW = NC * NS
    assert B % NW == 0 and D == 64
    bw = B // NW
    CH = 256
    assert bw % CH == 0
    ncb = bw // CH
    T = L * ncb
    assert T % 2 == 0

    @functools.partial(
        pl.kernel,
        mesh=plsc.VectorSubcoreMesh(core_axis_name="c", subcore_axis_name="s"),
        compiler_params=pltpu.CompilerParams(needs_layout_passes=False),
        out_type=jax.ShapeDtypeStruct((L, D, B), jnp.float32),
        scratch_types=[
            pltpu.VMEM((2, CH), jnp.int32),
            pltpu.VMEM((CH,), jnp.int32),
            pltpu.VMEM((CH,), jnp.int32),
            pltpu.VMEM((2, CH), jnp.int32),
            pltpu.VMEM((2, CH, 2 * D), jnp.float32),
            pltpu.VMEM((2, D, CH), jnp.float32),
            pltpu.VMEM((16, 16), jnp.int32),
        ]
        + [pltpu.SemaphoreType.DMA] * 6,
    )
    def ga(idx_t, pairs, out_t, idxb, pairv0, pairv1, colsel, rows_v, ot, rot_v, *sems):
        pairv = (pairv0, pairv1)
        semi = sems[0:2]
        semg = sems[2:4]
        semo = sems[4:6]
        wid = lax.axis_index("s") * NC + lax.axis_index("c")
        b_base = wid * bw
        iota = lax.iota(jnp.int32, 16)

        def idx_src(t):
            return idx_t.at[t // ncb, pl.ds(b_base + (t % ncb) * CH, CH)]

        def out_dst(t):
            return out_t.at[t // ncb, :, pl.ds(b_base + (t % ncb) * CH, CH)]

        def compute_and_fire(t, b):
            @plsc.parallel_loop(0, CH // 16, unroll=4)
            def _g(g):
                v = idxb[b, pl.ds(g * 16, 16)]
                pairv[b][pl.ds(g * 16, 16)] = lax.shift_right_logical(v, 1)
                colsel[b, pl.ds(g * 16, 16)] = lax.shift_left(v & 1, 6)

            pltpu.async_copy(pairs.at[pairv[b]], rows_v.at[b], semg[b])

        def select_transpose(b):
            @plsc.parallel_loop(0, CH // 16, unroll=2)
            def _gq(g):
                tokv = iota + g * 16
                cbase = colsel[b, pl.ds(g * 16, 16)]

                @plsc.parallel_loop(0, 16, unroll=4)
                def _k(k):
                    rot = rot_v[k, :]
                    ccol = cbase + rot
                    for di in range(D // 16):
                        drot = rot + di * 16
                        val = plsc.load_gather(
                            rows_v.at[b], [tokv, ccol + di * 16]
                        )
                        plsc.store_scatter(ot.at[b], [drot, tokv], val)

        for k in range(16):
            rot_v[k, :] = (iota + k) & 15

        pltpu.sync_copy(idx_src(0), idxb.at[0])
        compute_and_fire(0, 0)

        @pl.loop(0, T, step=2)
        def _outer(t0):
            for j in range(2):
                t = t0 + j
                b = j
                b1 = 1 - j

                def stage_next():
                    pltpu.async_copy(idx_src(t + 1), idxb.at[b1], semi[b1])

                if j == 0:
                    stage_next()
                else:
                    @pl.when(t + 1 < T)
                    def _():
                        stage_next()

                pltpu.make_async_copy(
                    pairs.at[pairv[b]], rows_v.at[b], semg[b]
                ).wait()

                @pl.when(t >= 2)
                def _():
                    pltpu.make_async_copy(
                        ot.at[b], out_dst(t - 2), semo[b]
                    ).wait()

                select_transpose(b)
                pltpu.async_copy(ot.at[b], out_dst(t), semo[b])

                def drain_and_fire():
                    pltpu.make_async_copy(
                        idx_src(t + 1), idxb.at[b1], semi[b1]
                    ).wait()
                    compute_and_fire(t + 1, b1)

                if j == 0:
                    drain_and_fire()
                else:
                    @pl.when(t + 1 < T)
                    def _():
                        drain_and_fire()

        for j in range(2):
            t = T - 2 + j
            pltpu.make_async_copy(ot.at[j], out_dst(t), semo[j]).wait()

    return ga


def kernel(input, table):
    B, L = input.shape
    V, D = table.shape
    try:
        info = plsc.get_sparse_core_info()
        NC, NS = info.num_cores, info.num_subcores
    except Exception:
        NC, NS = 2, 16
    idx_t = input.T.astype(jnp.int32)
    pairs = jnp.reshape(table, (V // 2, 2 * D))
    out_t = _build_gather(B, L, V, D, NC, NS)(idx_t, pairs)
    return out_t.transpose(2, 0, 1)


# speedup vs baseline: 5.0715x; 1.5151x over previous
import functools

import jax
import jax.numpy as jnp
from jax import lax
from jax.experimental import pallas as pl
from jax.experimental.pallas import tpu as pltpu
from jax.experimental.pallas import tpu_sc as plsc


def _build_gather(B, L, V, D, NC, NS):
    NW = NC * NS
    assert B % NW == 0 and D == 64
    bw = B // NW
    CH = 256
    assert bw % CH == 0
    ncb = bw // CH
    T = L * ncb
    assert T % 2 == 0

    @functools.partial(
        pl.kernel,
        mesh=plsc.VectorSubcoreMesh(core_axis_name="c", subcore_axis_name="s"),
        compiler_params=pltpu.CompilerParams(needs_layout_passes=False),
        out_type=jax.ShapeDtypeStruct((L, D, B), jnp.float32),
        scratch_types=[
            pltpu.VMEM((2, CH), jnp.int32),
            pltpu.VMEM((CH,), jnp.int32),
            pltpu.VMEM((CH,), jnp.int32),
            pltpu.VMEM((2, CH), jnp.int32),
            pltpu.VMEM((2, CH, 2 * D), jnp.float32),
            pltpu.VMEM((2, D, CH), jnp.float32),
            pltpu.VMEM((16, 16), jnp.int32),
        ]
        + [pltpu.SemaphoreType.DMA] * 6,
    )
    def ga(idx_t, pairs, out_t, idxb, pairv0, pairv1, colsel, rows_v, ot, rot_v, *sems):
        pairv = (pairv0, pairv1)
        semi = sems[0:2]
        semg = sems[2:4]
        semo = sems[4:6]
        wid = lax.axis_index("s") * NC + lax.axis_index("c")
        b_base = wid * bw
        iota = lax.iota(jnp.int32, 16)

        def idx_src(t):
            return idx_t.at[t // ncb, pl.ds(b_base + (t % ncb) * CH, CH)]

        def out_dst(t):
            return out_t.at[t // ncb, :, pl.ds(b_base + (t % ncb) * CH, CH)]

        def compute_and_fire(t, b):
            @plsc.parallel_loop(0, CH // 16, unroll=4)
            def _g(g):
                v = idxb[b, pl.ds(g * 16, 16)]
                pairv[b][pl.ds(g * 16, 16)] = lax.shift_right_logical(v, 1)
                colsel[b, pl.ds(g * 16, 16)] = lax.shift_left(v & 1, 6)

            pltpu.async_copy(pairs.at[pairv[b]], rows_v.at[b], semg[b])

        def select_transpose(b):
            @plsc.parallel_loop(0, CH // 16, unroll=2)
            def _gq(g):
                tokv = iota + g * 16
                cbase = colsel[b, pl.ds(g * 16, 16)]

                @plsc.parallel_loop(0, 16, unroll=4)
                def _k(k):
                    rot = rot_v[k, :]
                    ccol = cbase + rot
                    for di in range(D // 16):
                        drot = rot + di * 16
                        val = plsc.load_gather(
                            rows_v.at[b], [tokv, ccol + di * 16]
                        )
                        plsc.store_scatter(ot.at[b], [drot, tokv], val)

        for k in range(16):
            rot_v[k, :] = (iota + k) & 15

        pltpu.sync_copy(idx_src(0), idxb.at[0])
        compute_and_fire(0, 0)

        @pl.loop(0, T, step=2)
        def _outer(t0):
            for j in range(2):
                t = t0 + j
                b = j
                b1 = 1 - j

                def stage_next():
                    pltpu.async_copy(idx_src(t + 1), idxb.at[b1], semi[b1])

                if j == 0:
                    stage_next()
                else:
                    @pl.when(t + 1 < T)
                    def _():
                        stage_next()

                pltpu.make_async_copy(
                    pairs.at[pairv[b]], rows_v.at[b], semg[b]
                ).wait()

                @pl.when(t >= 2)
                def _():
                    pltpu.make_async_copy(
                        ot.at[b], out_dst(t - 2), semo[b]
                    ).wait()

                select_transpose(b)
                pltpu.async_copy(ot.at[b], out_dst(t), semo[b])

                def drain_and_fire():
                    pltpu.make_async_copy(
                        idx_src(t + 1), idxb.at[b1], semi[b1]
                    ).wait()
                    compute_and_fire(t + 1, b1)

                if j == 0:
                    drain_and_fire()
                else:
                    @pl.when(t + 1 < T)
                    def _():
                        drain_and_fire()

        for j in range(2):
            t = T - 2 + j
            pltpu.make_async_copy(ot.at[j], out_dst(t), semo[j]).wait()

    return ga


def _build_pairs(V, D, NC, NS):
    NW = NC * NS
    assert D == 64
    NFULL = V // 128
    TAIL = V - NFULL * 128
    assert TAIL in (0, 64)
    n_i = (NFULL + NW - 1) // NW
    n_up = n_i + (n_i % 2)

    @functools.partial(
        pl.kernel,
        mesh=plsc.VectorSubcoreMesh(core_axis_name="c", subcore_axis_name="s"),
        compiler_params=pltpu.CompilerParams(needs_layout_passes=False),
        out_type=jax.ShapeDtypeStruct((V // 2, 2 * D), jnp.float32),
        scratch_types=[
            pltpu.VMEM((2, D, 129), jnp.float32),
            pltpu.VMEM((2, D, 128), jnp.float32),
            pltpu.VMEM((16, 16), jnp.int32),
        ]
        + [pltpu.SemaphoreType.DMA] * 4,
    )
    def tr(table_t, tail_pairs, pairs, tb, tt, rot_v, *sems):
        semi = sems[:2]
        semo = sems[2:]
        wid = lax.axis_index("s") * NC + lax.axis_index("c")
        iota = lax.iota(jnp.int32, 16)
        two_iota = iota * 2

        for k in range(16):
            rot_v[k, :] = (iota + k) & 15

        def blk(i):
            return i * NW + wid

        def in_src(i):
            return table_t.at[:, pl.ds(blk(i) * 128, 128)]

        def in_dst(b):
            return tb.at[b, :, pl.ds(0, 128)]

        def out_dst(i):
            return pairs.at[pl.ds(blk(i) * 64, 64), :]

        def start_in(i, b):
            pltpu.async_copy(in_src(i), in_dst(b), semi[b])

        def compute(b):
            @plsc.parallel_loop(0, 16, unroll=2)
            def _k(k):
                rot = rot_v[k, :]
                for qi in range(8):
                    q0 = qi * 16
                    h = 0 if q0 < D else 1
                    d0 = q0 - D * h
                    rowg = rot + d0
                    colv = rot + q0
                    for pi in range(4):
                        p0 = pi * 16
                        val = plsc.load_gather(
                            tb.at[b], [rowg, two_iota + (2 * p0 + h)]
                        )
                        plsc.store_scatter(tt.at[b], [iota + p0, colv], val)

        start_in(0, 0)

        @pl.loop(0, n_up, step=2)
        def _outer(t0):
            for j in range(2):
                i = t0 + j
                b = j
                b1 = 1 - j

                @pl.when((i + 1 < n_i) & (blk(i + 1) < NFULL))
                def _():
                    start_in(i + 1, b1)

                @pl.when(blk(i) < NFULL)
                def _():
                    pltpu.make_async_copy(in_src(i), in_dst(b), semi[b]).wait()

                    @pl.when(i >= 2)
                    def _():
                        pltpu.make_async_copy(
                            tt.at[b], out_dst(i - 2), semo[b]
                        ).wait()

                    compute(b)
                    pltpu.async_copy(tt.at[b], out_dst(i), semo[b])

        for j in range(2):
            i_last = n_i - 2 + j

            @pl.when((i_last >= 0) & (blk(i_last) < NFULL))
            def _():
                pltpu.make_async_copy(
                    tt.at[i_last % 2], out_dst(i_last), semo[i_last % 2]
                ).wait()

        if TAIL:
            @pl.when(wid == NW - 1)
            def _tail():
                pltpu.sync_copy(
                    tail_pairs, pairs.at[pl.ds(NFULL * 64, TAIL // 2), :]
                )

    return tr


def kernel(input, table):
    B, L = input.shape
    V, D = table.shape
    try:
        info = plsc.get_sparse_core_info()
        NC, NS = info.num_cores, info.num_subcores
    except Exception:
        NC, NS = 2, 16
    idx_t = input.T.astype(jnp.int32)
    table_t = table.T
    nfull = V // 128
    tail_pairs = table[nfull * 128:, :].reshape(-1, 2 * D)
    pairs = _build_pairs(V, D, NC, NS)(table_t, tail_pairs)
    out_t = _build_gather(B, L, V, D, NC, NS)(idx_t, pairs)
    return out_t.transpose(2, 0, 1)


# speedup vs baseline: 5.0782x; 1.0013x over previous
import functools

import jax
import jax.numpy as jnp
from jax import lax
from jax.experimental import pallas as pl
from jax.experimental.pallas import tpu as pltpu
from jax.experimental.pallas import tpu_sc as plsc


def _build_gather(B, L, V, D, NC, NS):
    NW = NC * NS
    assert B % NW == 0 and D == 64
    bw = B // NW
    CH = 256
    assert bw % CH == 0
    ncb = bw // CH
    T = L * ncb
    assert T % 2 == 0

    @functools.partial(
        pl.kernel,
        mesh=plsc.VectorSubcoreMesh(core_axis_name="c", subcore_axis_name="s"),
        compiler_params=pltpu.CompilerParams(needs_layout_passes=False),
        out_type=jax.ShapeDtypeStruct((L, D, B), jnp.float32),
        scratch_types=[
            pltpu.VMEM((2, CH), jnp.int32),
            pltpu.VMEM((CH,), jnp.int32),
            pltpu.VMEM((CH,), jnp.int32),
            pltpu.VMEM((2, CH), jnp.int32),
            pltpu.VMEM((2, CH, 2 * D), jnp.float32),
            pltpu.VMEM((2, D, CH), jnp.float32),
            pltpu.VMEM((16, 16), jnp.int32),
        ]
        + [pltpu.SemaphoreType.DMA] * 6,
    )
    def ga(idx_t, pairs, out_t, idxb, pairv0, pairv1, colsel, rows_v, ot, rot_v, *sems):
        pairv = (pairv0, pairv1)
        semi = sems[0:2]
        semg = sems[2:4]
        semo = sems[4:6]
        wid = lax.axis_index("s") * NC + lax.axis_index("c")
        b_base = wid * bw
        iota = lax.iota(jnp.int32, 16)

        def idx_src(t):
            return idx_t.at[t // ncb, pl.ds(b_base + (t % ncb) * CH, CH)]

        def out_dst(t):
            return out_t.at[t // ncb, :, pl.ds(b_base + (t % ncb) * CH, CH)]

        def compute_and_fire(t, b):
            @plsc.parallel_loop(0, CH // 16, unroll=4)
            def _g(g):
                v = idxb[b, pl.ds(g * 16, 16)]
                pairv[b][pl.ds(g * 16, 16)] = lax.shift_right_logical(v, 1)
                colsel[b, pl.ds(g * 16, 16)] = lax.shift_left(v & 1, 6)

            pltpu.async_copy(pairs.at[pairv[b]], rows_v.at[b], semg[b])

        def select_transpose(b):
            @plsc.parallel_loop(0, CH // 16, unroll=4)
            def _gq(g):
                tokv = iota + g * 16
                cbase = colsel[b, pl.ds(g * 16, 16)]

                @plsc.parallel_loop(0, 16, unroll=4)
                def _k(k):
                    rot = rot_v[k, :]
                    ccol = cbase + rot
                    for di in range(D // 16):
                        drot = rot + di * 16
                        val = plsc.load_gather(
                            rows_v.at[b], [tokv, ccol + di * 16]
                        )
                        plsc.store_scatter(ot.at[b], [drot, tokv], val)

        for k in range(16):
            rot_v[k, :] = (iota + k) & 15

        pltpu.sync_copy(idx_src(0), idxb.at[0])
        compute_and_fire(0, 0)

        @pl.loop(0, T, step=2)
        def _outer(t0):
            for j in range(2):
                t = t0 + j
                b = j
                b1 = 1 - j

                def stage_next():
                    pltpu.async_copy(idx_src(t + 1), idxb.at[b1], semi[b1])

                if j == 0:
                    stage_next()
                else:
                    @pl.when(t + 1 < T)
                    def _():
                        stage_next()

                pltpu.make_async_copy(
                    pairs.at[pairv[b]], rows_v.at[b], semg[b]
                ).wait()

                @pl.when(t >= 2)
                def _():
                    pltpu.make_async_copy(
                        ot.at[b], out_dst(t - 2), semo[b]
                    ).wait()

                select_transpose(b)
                pltpu.async_copy(ot.at[b], out_dst(t), semo[b])

                def drain_and_fire():
                    pltpu.make_async_copy(
                        idx_src(t + 1), idxb.at[b1], semi[b1]
                    ).wait()
                    compute_and_fire(t + 1, b1)

                if j == 0:
                    drain_and_fire()
                else:
                    @pl.when(t + 1 < T)
                    def _():
                        drain_and_fire()

        for j in range(2):
            t = T - 2 + j
            pltpu.make_async_copy(ot.at[j], out_dst(t), semo[j]).wait()

    return ga


def _build_pairs(V, D, NC, NS):
    NW = NC * NS
    assert D == 64
    NFULL = V // 128
    TAIL = V - NFULL * 128
    assert TAIL in (0, 64)
    n_i = (NFULL + NW - 1) // NW
    n_up = n_i + (n_i % 2)

    @functools.partial(
        pl.kernel,
        mesh=plsc.VectorSubcoreMesh(core_axis_name="c", subcore_axis_name="s"),
        compiler_params=pltpu.CompilerParams(needs_layout_passes=False),
        out_type=jax.ShapeDtypeStruct((V // 2, 2 * D), jnp.float32),
        scratch_types=[
            pltpu.VMEM((2, D, 129), jnp.float32),
            pltpu.VMEM((2, D, 128), jnp.float32),
            pltpu.VMEM((16, 16), jnp.int32),
        ]
        + [pltpu.SemaphoreType.DMA] * 4,
    )
    def tr(table_t, tail_pairs, pairs, tb, tt, rot_v, *sems):
        semi = sems[:2]
        semo = sems[2:]
        wid = lax.axis_index("s") * NC + lax.axis_index("c")
        iota = lax.iota(jnp.int32, 16)
        two_iota = iota * 2

        for k in range(16):
            rot_v[k, :] = (iota + k) & 15

        def blk(i):
            return i * NW + wid

        def in_src(i):
            return table_t.at[:, pl.ds(blk(i) * 128, 128)]

        def in_dst(b):
            return tb.at[b, :, pl.ds(0, 128)]

        def out_dst(i):
            return pairs.at[pl.ds(blk(i) * 64, 64), :]

        def start_in(i, b):
            pltpu.async_copy(in_src(i), in_dst(b), semi[b])

        def compute(b):
            @plsc.parallel_loop(0, 16, unroll=2)
            def _k(k):
                rot = rot_v[k, :]
                for qi in range(8):
                    q0 = qi * 16
                    h = 0 if q0 < D else 1
                    d0 = q0 - D * h
                    rowg = rot + d0
                    colv = rot + q0
                    for pi in range(4):
                        p0 = pi * 16
                        val = plsc.load_gather(
                            tb.at[b], [rowg, two_iota + (2 * p0 + h)]
                        )
                        plsc.store_scatter(tt.at[b], [iota + p0, colv], val)

        start_in(0, 0)

        @pl.loop(0, n_up, step=2)
        def _outer(t0):
            for j in range(2):
                i = t0 + j
                b = j
                b1 = 1 - j

                @pl.when((i + 1 < n_i) & (blk(i + 1) < NFULL))
                def _():
                    start_in(i + 1, b1)

                @pl.when(blk(i) < NFULL)
                def _():
                    pltpu.make_async_copy(in_src(i), in_dst(b), semi[b]).wait()

                    @pl.when(i >= 2)
                    def _():
                        pltpu.make_async_copy(
                            tt.at[b], out_dst(i - 2), semo[b]
                        ).wait()

                    compute(b)
                    pltpu.async_copy(tt.at[b], out_dst(i), semo[b])

        for j in range(2):
            i_last = n_i - 2 + j

            @pl.when((i_last >= 0) & (blk(i_last) < NFULL))
            def _():
                pltpu.make_async_copy(
                    tt.at[i_last % 2], out_dst(i_last), semo[i_last % 2]
                ).wait()

        if TAIL:
            @pl.when(wid == NW - 1)
            def _tail():
                pltpu.sync_copy(
                    tail_pairs, pairs.at[pl.ds(NFULL * 64, TAIL // 2), :]
                )

    return tr


def kernel(input, table):
    B, L = input.shape
    V, D = table.shape
    try:
        info = plsc.get_sparse_core_info()
        NC, NS = info.num_cores, info.num_subcores
    except Exception:
        NC, NS = 2, 16
    idx_t = input.T.astype(jnp.int32)
    table_t = table.T
    nfull = V // 128
    tail_pairs = table[nfull * 128:, :].reshape(-1, 2 * D)
    pairs = _build_pairs(V, D, NC, NS)(table_t, tail_pairs)
    out_t = _build_gather(B, L, V, D, NC, NS)(idx_t, pairs)
    return out_t.transpose(2, 0, 1)
